# baseline (speedup 1.0000x reference)
import functools

import jax
import jax.numpy as jnp
from jax import lax
from jax.experimental import pallas as pl
from jax.experimental.pallas import tpu as pltpu
from jax.experimental.pallas import tpu_sc as plsc

_NC, _NS = 2, 16
_NW = _NC * _NS
_N_PTS = 4096
_CH = 64
_CHUNK = _N_PTS // _NW
_LEVELS = ((7, 256, 256), (5, 128, 128), (3, 64, 64))
_D_TOT = sum(k * k for k, _, _ in _LEVELS)


def _sc_gather(px, py, t0, t1, t2):
    mesh = plsc.VectorSubcoreMesh(core_axis_name="c", subcore_axis_name="s")

    @functools.partial(
        pl.kernel,
        mesh=mesh,
        compiler_params=pltpu.CompilerParams(use_tc_tiling_on_sc=False),
        out_type=jax.ShapeDtypeStruct((_D_TOT, _N_PTS, _CH), jnp.float32),
        scratch_types=[
            pltpu.VMEM((_CHUNK,), jnp.float32),
            pltpu.VMEM((_CHUNK,), jnp.float32),
            pltpu.VMEM((_CHUNK,), jnp.float32),
            pltpu.VMEM((_CHUNK,), jnp.float32),
            pltpu.VMEM((_CHUNK,), jnp.int32),
            pltpu.VMEM((_CHUNK, _CH), jnp.float32),
            pltpu.SemaphoreType.DMA,
        ],
    )
    def k(px_hbm, py_hbm, t0_hbm, t1_hbm, t2_hbm, g_hbm,
          px_v, py_v, xf_v, yf_v, idx_v, rows_v, sem):
        wid = lax.axis_index("s") * _NC + lax.axis_index("c")
        base = wid * _CHUNK
        pltpu.sync_copy(px_hbm.at[pl.ds(base, _CHUNK)], px_v)
        pltpu.sync_copy(py_hbm.at[pl.ds(base, _CHUNK)], py_v)

        dbase = 0
        for tab, (kk, h, w) in zip((t0_hbm, t1_hbm, t2_hbm), _LEVELS):
            half = kk // 2
            wm1 = float(w - 1)
            hm1 = float(h - 1)
            for j in range(_CHUNK // 16):
                s = pl.ds(j * 16, 16)
                xf_v[s] = jnp.clip(px_v[s] * wm1, 0.0, wm1)
                yf_v[s] = jnp.clip(py_v[s] * hm1, 0.0, hm1)

            def tap_body(t, carry, tab=tab, kk=kk, h=h, w=w, half=half,
                         dbase=dbase, wm1=wm1, hm1=hm1):
                dyf = (t // kk - half).astype(jnp.float32)
                dxf = (t % kk - half).astype(jnp.float32)
                for j in range(_CHUNK // 16):
                    s = pl.ds(j * 16, 16)
                    tx = jnp.clip(xf_v[s] + dxf, 0.0, wm1).astype(jnp.int32)
                    ty = jnp.clip(yf_v[s] + dyf, 0.0, hm1).astype(jnp.int32)
                    idx_v[s] = ty * w + tx
                pltpu.async_copy(tab.at[idx_v], rows_v, sem).wait()
                pltpu.sync_copy(rows_v, g_hbm.at[dbase + t, pl.ds(base, _CHUNK), :])
                return carry

            lax.fori_loop(0, kk * kk, tap_body, 0)
            dbase += kk * kk

    return k(px, py, t0, t1, t2)


def _tc_transpose(x):
    m = x.shape[1]
    blk = 1024

    def body(x_ref, o_ref):
        o_ref[...] = jnp.transpose(x_ref[...])

    return pl.pallas_call(
        body,
        grid=(m // blk,),
        in_specs=[pl.BlockSpec((_CH, blk), lambda i: (0, i))],
        out_specs=pl.BlockSpec((blk, _CH), lambda i: (i, 0)),
        out_shape=jax.ShapeDtypeStruct((m, _CH), jnp.float32),
    )(x)


def _tc_project(g_flat, w_t, b2):
    blk = 2048
    grid = (g_flat.shape[1] // blk,)

    def body(g_ref, w_ref, b_ref, o_ref):
        acc = lax.dot_general(g_ref[...], w_ref[...],
                              (((0,), (0,)), ((), ())),
                              preferred_element_type=jnp.float32)
        o_ref[...] = acc + b_ref[...]

    return pl.pallas_call(
        body,
        grid=grid,
        in_specs=[
            pl.BlockSpec((_D_TOT, blk), lambda i: (0, i)),
            pl.BlockSpec((_D_TOT, 32), lambda i: (0, 0)),
            pl.BlockSpec((1, 32), lambda i: (0, 0)),
        ],
        out_specs=pl.BlockSpec((blk, 32), lambda i: (i, 0)),
        out_shape=jax.ShapeDtypeStruct((g_flat.shape[1], 32), jnp.float32),
    )(g_flat, w_t, b2)


def kernel(points, features_0, features_1, features_2, W_fc, b_fc):
    px = points[0, :, 0]
    py = points[0, :, 1]
    tables = []
    for feat, (_, h, w) in zip((features_0, features_1, features_2), _LEVELS):
        tables.append(_tc_transpose(feat[0].reshape(_CH, h * w)))
    g = _sc_gather(px, py, *tables)
    g_flat = g.reshape(_D_TOT, _N_PTS * _CH)
    proj = _tc_project(g_flat, jnp.transpose(W_fc), b_fc.reshape(1, 32))
    return proj.reshape(1, _CH, _N_PTS, W_fc.shape[0])


# speedup vs baseline: 2.2210x; 2.2210x over previous
import functools

import jax
import jax.numpy as jnp
from jax import lax
from jax.experimental import pallas as pl
from jax.experimental.pallas import tpu as pltpu
from jax.experimental.pallas import tpu_sc as plsc

_NC, _NS = 2, 16
_NW = _NC * _NS
_N_PTS = 4096
_CH = 64
_CHUNK = _N_PTS // _NW
_H2 = _CHUNK // 2
_LEVELS = ((7, 256, 256), (5, 128, 128), (3, 64, 64))
_D_TOT = sum(k * k for k, _, _ in _LEVELS)


def _sc_gather(px, py, t0, t1, t2):
    mesh = plsc.VectorSubcoreMesh(core_axis_name="c", subcore_axis_name="s")

    @functools.partial(
        pl.kernel,
        mesh=mesh,
        compiler_params=pltpu.CompilerParams(use_tc_tiling_on_sc=False),
        out_type=jax.ShapeDtypeStruct((_D_TOT, _N_PTS * _CH // 128, 128), jnp.float32),
        scratch_types=[
            pltpu.VMEM((_CHUNK,), jnp.float32),
            pltpu.VMEM((_CHUNK,), jnp.float32),
            pltpu.VMEM((_CHUNK,), jnp.float32),
            pltpu.VMEM((_CHUNK,), jnp.float32),
            pltpu.VMEM((_CHUNK,), jnp.int32),
            pltpu.VMEM((_CHUNK,), jnp.int32),
            pltpu.VMEM((_CHUNK, _CH), jnp.float32),
            pltpu.VMEM((_CHUNK, _CH), jnp.float32),
            pltpu.SemaphoreType.DMA,
            pltpu.SemaphoreType.DMA,
            pltpu.SemaphoreType.DMA,
            pltpu.SemaphoreType.DMA,
        ],
    )
    def k(px_hbm, py_hbm, t0_hbm, t1_hbm, t2_hbm, g_hbm,
          px_v, py_v, xf_v, yf_v, ia_v, ib_v, ra_v, rb_v,
          gsa, gsb, ssa, ssb):
        wid = lax.axis_index("s") * _NC + lax.axis_index("c")
        base = wid * _CHUNK

        pltpu.sync_copy(px_hbm.at[pl.ds(base, _CHUNK)], px_v)
        pltpu.sync_copy(py_hbm.at[pl.ds(base, _CHUNK)], py_v)

        dbase = 0
        for tab, (kk, h, w) in zip((t0_hbm, t1_hbm, t2_hbm), _LEVELS):
            nt = kk * kk
            half = kk // 2
            wm1 = float(w - 1)
            hm1 = float(h - 1)
            for j in range(_CHUNK // 16):
                s = pl.ds(j * 16, 16)
                xf_v[s] = jnp.clip(px_v[s] * wm1, 0.0, wm1)
                yf_v[s] = jnp.clip(py_v[s] * hm1, 0.0, hm1)

            def idx_into(t, dst, kk=kk, w=w, half=half, wm1=wm1, hm1=hm1):
                t = jnp.int32(t)
                dyf = (t // kk - half).astype(jnp.float32)
                dxf = (t % kk - half).astype(jnp.float32)
                for j in range(_CHUNK // 16):
                    s = pl.ds(j * 16, 16)
                    tx = jnp.clip(xf_v[s] + dxf, 0.0, wm1).astype(jnp.int32)
                    ty = jnp.clip(yf_v[s] + dyf, 0.0, hm1).astype(jnp.int32)
                    dst[s] = ty * w + tx

            def scatter_start(t, rows, sem, dbase=dbase):
                c1 = pltpu.async_copy(
                    rows.at[pl.ds(0, _H2), :],
                    g_hbm.at[dbase + t, pl.ds(wid * _H2, _H2), pl.ds(0, _CH)],
                    sem)
                c2 = pltpu.async_copy(
                    rows.at[pl.ds(_H2, _H2), :],
                    g_hbm.at[dbase + t, pl.ds(wid * _H2, _H2), pl.ds(_CH, _CH)],
                    sem)
                return c1, c2

            idx_into(0, ia_v)
            pltpu.async_copy(tab.at[ia_v], ra_v, gsa)

            def pair_body(i, carry, tab=tab, dbase=dbase):
                ta = 2 * i
                tb = 2 * i + 1
                idx_into(tb, ib_v)
                pltpu.async_copy(tab.at[ib_v], rb_v, gsb)
                pltpu.make_async_copy(tab.at[ia_v], ra_v, gsa).wait()
                sa1, sa2 = scatter_start(ta, ra_v, ssa)
                idx_into(ta + 2, ia_v)
                sa1.wait()
                sa2.wait()
                pltpu.async_copy(tab.at[ia_v], ra_v, gsa)
                pltpu.make_async_copy(tab.at[ib_v], rb_v, gsb).wait()
                sb1, sb2 = scatter_start(tb, rb_v, ssb)
                sb1.wait()
                sb2.wait()
                return carry

            lax.fori_loop(0, (nt - 1) // 2, pair_body, 0)

            t_last = jnp.int32(nt - 1)
            pltpu.make_async_copy(tab.at[ia_v], ra_v, gsa).wait()
            se1, se2 = scatter_start(t_last, ra_v, ssa)
            se1.wait()
            se2.wait()

            dbase += nt

    return k(px, py, t0, t1, t2)


def _tc_project(g3, w_t, b2):
    nb = 16
    grid = (g3.shape[1] // nb,)

    def body(g_ref, w_ref, b_ref, o_ref):
        wb = w_ref[...].astype(jnp.bfloat16)
        for j in range(nb):
            acc = lax.dot_general(g_ref[:, j, :].astype(jnp.bfloat16), wb,
                                  (((0,), (0,)), ((), ())),
                                  preferred_element_type=jnp.float32)
            o_ref[pl.ds(j * 128, 128), :] = acc + b_ref[...]

    return pl.pallas_call(
        body,
        grid=grid,
        in_specs=[
            pl.BlockSpec((_D_TOT, nb, 128), lambda i: (0, i, 0)),
            pl.BlockSpec((_D_TOT, 32), lambda i: (0, 0)),
            pl.BlockSpec((1, 32), lambda i: (0, 0)),
        ],
        out_specs=pl.BlockSpec((nb * 128, 32), lambda i: (i, 0)),
        out_shape=jax.ShapeDtypeStruct((_N_PTS * _CH, 32), jnp.float32),
    )(g3, w_t, b2)


def kernel(points, features_0, features_1, features_2, W_fc, b_fc):
    pxy = points[0].reshape(_NW, _CHUNK // 2, 2, 2)
    pxy = jnp.transpose(pxy, (0, 2, 1, 3)).reshape(_N_PTS, 2)
    px = pxy[:, 0]
    py = pxy[:, 1]
    tables = []
    for feat, (_, h, w) in zip((features_0, features_1, features_2), _LEVELS):
        tables.append(jnp.transpose(feat[0].reshape(_CH, h * w)))
    g3 = _sc_gather(px, py, *tables)
    proj = _tc_project(g3, jnp.transpose(W_fc), b_fc.reshape(1, 32))
    return proj.reshape(1, _CH, _N_PTS, W_fc.shape[0])
